# baseline (speedup 1.0000x reference)
import functools

import jax
import jax.numpy as jnp
from jax import lax
from jax.experimental import pallas as pl
from jax.experimental.pallas import tpu as pltpu
from jax.experimental.pallas import tpu_sc as plsc

NC, NS = 2, 16
NW = NC * NS
B = 1024
S1_PER = 10
S2_PER = 25
D = 256
S1 = B * S1_PER
S2 = S1 * S2_PER
NB_W = B // NW
S1_W = NB_W * S1_PER
S2_W = S1_W * S2_PER
GP = 8
G_ROWS = GP * S2_PER
NG = S1_W // GP
N_CLS = 128
Q_PAR = S1_W // 4
Q_S2 = Q_PAR * S2_PER
H1_CHUNK = 80

_sc_mesh = plsc.VectorSubcoreMesh(
    core_axis_name="c", subcore_axis_name="s", num_cores=NC, num_subcores=NS
)


def _sc_body(nodes_hbm, adj_hbm, feat_hbm, c1_hbm, c2_hbm,
             h0_hbm, h1_hbm, m2_hbm,
             nodes_v, nd4_v, nm4_v, adj0_v, c1_v, s1_v, s1d4_v, s1m4_v,
             adj1_v, c2_v, s2_v,
             buf0, buf1, stage, sem0, sem1):
    w = lax.axis_index("s") * NC + lax.axis_index("c")
    iota = lax.iota(jnp.int32, 16)


    pltpu.sync_copy(nodes_hbm.at[pl.ds(w * NB_W, NB_W)], nodes_v)

    def nsplit_body(i, carry):
        nv = nodes_v[pl.ds(i * 16, 16)]
        nd4_v[pl.ds(i * 16, 16)] = nv // 4
        nm4_v[pl.ds(i * 16, 16)] = (nv % 4) * 32
        return carry

    lax.fori_loop(0, NB_W // 16, nsplit_body, 0)
    pltpu.async_copy(adj_hbm.at[nd4_v], adj0_v, sem0).wait()
    pltpu.sync_copy(c1_hbm.at[pl.ds(w * S1_W, S1_W)], c1_v)

    def s1_body(i, carry):
        e = i * 16 + iota
        r = e // S1_PER
        cv = c1_v[pl.ds(i * 16, 16)] + plsc.load_gather(nm4_v, [r])
        s1_v[pl.ds(i * 16, 16)] = plsc.load_gather(adj0_v, [r, cv])
        return carry

    lax.fori_loop(0, S1_W // 16, s1_body, 0)

    def s1split_body(i, carry):
        sv = s1_v[pl.ds(i * 16, 16)]
        s1d4_v[pl.ds(i * 16, 16)] = sv // 4
        s1m4_v[pl.ds(i * 16, 16)] = (sv % 4) * 32
        return carry

    lax.fori_loop(0, S1_W // 16, s1split_body, 0)

    def s2_quarter(q, carry):
        pltpu.async_copy(adj_hbm.at[s1d4_v.at[pl.ds(q * Q_PAR, Q_PAR)]],
                         adj1_v, sem0).wait()
        pltpu.sync_copy(c2_hbm.at[pl.ds(w * S2_W + q * Q_S2, Q_S2)], c2_v)

        def s2_body(i, inner):
            e = q * Q_S2 + i * 16 + iota
            r = e // S2_PER
            cv = c2_v[pl.ds(i * 16, 16)] + plsc.load_gather(s1m4_v, [r])
            s2_v[pl.ds(q * Q_S2 + i * 16, 16)] = plsc.load_gather(
                adj1_v, [r - q * Q_PAR, cv])
            return inner

        lax.fori_loop(0, Q_S2 // 16, s2_body, 0)
        return carry

    lax.fori_loop(0, 4, s2_quarter, 0)

    pltpu.async_copy(feat_hbm.at[nodes_v], buf0.at[pl.ds(0, NB_W)], sem0).wait()
    pltpu.sync_copy(buf0.at[pl.ds(0, NB_W)], h0_hbm.at[pl.ds(w * NB_W, NB_W)])

    for k in range(S1_W // H1_CHUNK):
        pltpu.async_copy(feat_hbm.at[s1_v.at[pl.ds(k * H1_CHUNK, H1_CHUNK)]],
                         buf0.at[pl.ds(0, H1_CHUNK)], sem0).wait()
        pltpu.sync_copy(buf0.at[pl.ds(0, H1_CHUNK)],
                        h1_hbm.at[pl.ds(w * S1_W + k * H1_CHUNK, H1_CHUNK)])

    def fire(g, buf, sem):
        off = g * G_ROWS
        pltpu.async_copy(feat_hbm.at[s2_v.at[pl.ds(off, 128)]],
                         buf.at[pl.ds(0, 128)], sem)
        pltpu.async_copy(feat_hbm.at[s2_v.at[pl.ds(off + 128, G_ROWS - 128)]],
                         buf.at[pl.ds(128, G_ROWS - 128)], sem)

    def drain(buf, sem):
        pltpu.make_async_copy(feat_hbm.at[pl.ds(0, G_ROWS)], buf, sem).wait()

    def process(g, buf):
        def par_body(p, carry):
            base = p * S2_PER

            def row_body(r, acc):
                row = base + r
                return tuple(acc[c] + buf[row, pl.ds(c * 16, 16)]
                             for c in range(16))

            acc0 = tuple(jnp.zeros((16,), jnp.float32) for _ in range(16))
            acc = lax.fori_loop(0, S2_PER, row_body, acc0)
            for c in range(16):
                stage[p, pl.ds(c * 16, 16)] = acc[c] * (1.0 / S2_PER)
            return carry

        lax.fori_loop(0, GP, par_body, 0)
        pltpu.sync_copy(stage, m2_hbm.at[pl.ds(w * S1_W + g * GP, GP)])

    fire(0, buf0, sem0)

    def m2_body(t, carry):
        g0 = 2 * t
        fire(g0 + 1, buf1, sem1)
        drain(buf0, sem0)
        process(g0, buf0)

        @pl.when(t < NG // 2 - 1)
        def _():
            fire(g0 + 2, buf0, sem0)

        drain(buf1, sem1)
        process(g0 + 1, buf1)
        return carry

    lax.fori_loop(0, NG // 2, m2_body, 0)


_sc_gather = functools.partial(
    pl.kernel,
    out_type=(
        jax.ShapeDtypeStruct((B, D), jnp.float32),
        jax.ShapeDtypeStruct((S1, D), jnp.float32),
        jax.ShapeDtypeStruct((S1, D), jnp.float32),
    ),
    mesh=_sc_mesh,
    compiler_params=pltpu.CompilerParams(needs_layout_passes=False, use_tc_tiling_on_sc=True),
    scratch_types=[
        pltpu.VMEM((NB_W,), jnp.int32),
        pltpu.VMEM((NB_W,), jnp.int32),
        pltpu.VMEM((NB_W,), jnp.int32),
        pltpu.VMEM((NB_W, 128), jnp.int32),
        pltpu.VMEM((S1_W,), jnp.int32),
        pltpu.VMEM((S1_W,), jnp.int32),
        pltpu.VMEM((S1_W,), jnp.int32),
        pltpu.VMEM((S1_W,), jnp.int32),
        pltpu.VMEM((Q_PAR, 128), jnp.int32),
        pltpu.VMEM((Q_S2,), jnp.int32),
        pltpu.VMEM((S2_W,), jnp.int32),
        pltpu.VMEM((G_ROWS, D), jnp.float32),
        pltpu.VMEM((G_ROWS, D), jnp.float32),
        pltpu.VMEM((GP, D), jnp.float32),
        pltpu.SemaphoreType.DMA,
        pltpu.SemaphoreType.DMA,
    ],
)(_sc_body)


T1_ROWS = 1280
T1_OUT = T1_ROWS // S1_PER
T1_STEPS = S1 // T1_ROWS


def _tc_body(h1_ref, m2_ref, h0_ref, w1_ref, w2_ref, fcw_ref, fcb_ref,
             out_ref, a1m_s, h1m_s):
    i = pl.program_id(0)
    h1b = h1_ref[...]
    w1 = w1_ref[...]
    x = (jnp.dot(h1b, w1[:D], preferred_element_type=jnp.float32)
         + jnp.dot(m2_ref[...], w1[D:], preferred_element_type=jnp.float32))
    x = jnp.maximum(x, 0.0)
    nrm = jnp.sqrt(jnp.sum(x * x, axis=1, keepdims=True))
    a1 = x / (nrm + 1e-8)
    a1m_s[pl.ds(i * T1_OUT, T1_OUT), :] = jnp.mean(
        a1.reshape(T1_OUT, S1_PER, D), axis=1)
    h1m_s[pl.ds(i * T1_OUT, T1_OUT), :] = jnp.mean(
        h1b.reshape(T1_OUT, S1_PER, D), axis=1)

    @pl.when(i == T1_STEPS - 1)
    def _():
        x0 = (jnp.dot(h0_ref[...], w1[:D], preferred_element_type=jnp.float32)
              + jnp.dot(h1m_s[...], w1[D:], preferred_element_type=jnp.float32))
        x0 = jnp.maximum(x0, 0.0)
        a0 = x0 / (jnp.sqrt(jnp.sum(x0 * x0, axis=1, keepdims=True)) + 1e-8)
        w2 = w2_ref[...]
        y = (jnp.dot(a0, w2[:D], preferred_element_type=jnp.float32)
             + jnp.dot(a1m_s[...], w2[D:], preferred_element_type=jnp.float32))
        y = jnp.maximum(y, 0.0)
        b0 = y / (jnp.sqrt(jnp.sum(y * y, axis=1, keepdims=True)) + 1e-8)
        out_ref[...] = (jnp.dot(b0, fcw_ref[...],
                                preferred_element_type=jnp.float32)
                        + fcb_ref[...])


def _tc(h1, m2, h0, W1, W2, fc_w, fc_b):
    return pl.pallas_call(
        _tc_body,
        grid=(T1_STEPS,),
        in_specs=[
            pl.BlockSpec((T1_ROWS, D), lambda i: (i, 0)),
            pl.BlockSpec((T1_ROWS, D), lambda i: (i, 0)),
            pl.BlockSpec((B, D), lambda i: (0, 0)),
            pl.BlockSpec((2 * D, D), lambda i: (0, 0)),
            pl.BlockSpec((2 * D, D), lambda i: (0, 0)),
            pl.BlockSpec((D, N_CLS), lambda i: (0, 0)),
            pl.BlockSpec((1, N_CLS), lambda i: (0, 0)),
        ],
        out_specs=pl.BlockSpec((B, N_CLS), lambda i: (0, 0)),
        out_shape=jax.ShapeDtypeStruct((B, N_CLS), jnp.float32),
        scratch_shapes=[
            pltpu.VMEM((B, D), jnp.float32),
            pltpu.VMEM((B, D), jnp.float32),
        ],
    )(h1, m2, h0, W1, W2, fc_w, fc_b.reshape(1, -1))


def kernel(nodes, features, adj, W1, W2, fc_w, fc_b):
    nodes = nodes.astype(jnp.int32)
    max_deg = adj.shape[1]
    adj = adj.astype(jnp.int32).reshape(-1, 128)
    key = jax.random.key(42)
    key, sub = jax.random.split(key)
    cols1 = jax.random.randint(sub, (B, S1_PER), 0, max_deg)
    key, sub = jax.random.split(key)
    cols2 = jax.random.randint(sub, (S1, S2_PER), 0, max_deg)
    c1 = cols1.reshape(-1).astype(jnp.int32)
    c2 = cols2.reshape(-1).astype(jnp.int32)

    h0, h1, m2 = _sc_gather(nodes, adj, features, c1, c2)
    return _tc(h1, m2, h0, W1, W2, fc_w, fc_b)


# speedup vs baseline: 1.0209x; 1.0209x over previous
import functools

import jax
import jax.numpy as jnp
from jax import lax
from jax.experimental import pallas as pl
from jax.experimental.pallas import tpu as pltpu
from jax.experimental.pallas import tpu_sc as plsc

NC, NS = 2, 16
NW = NC * NS
B = 1024
S1_PER = 10
S2_PER = 25
D = 256
S1 = B * S1_PER
S2 = S1 * S2_PER
NB_W = B // NW
S1_W = NB_W * S1_PER
S2_W = S1_W * S2_PER
GP = 8
G_ROWS = GP * S2_PER
NG = S1_W // GP
N_CLS = 128
Q_PAR = S1_W // 4
Q_S2 = Q_PAR * S2_PER
H1_CHUNK = 80

_sc_mesh = plsc.VectorSubcoreMesh(
    core_axis_name="c", subcore_axis_name="s", num_cores=NC, num_subcores=NS
)


def _sc_body(nodes_hbm, adj_hbm, feat_hbm, c1_hbm, c2_hbm,
             h0_hbm, h1_hbm, m2_hbm,
             nodes_v, nd4_v, nm4_v, adj0_v, c1_v, s1_v, s1d4_v, s1m4_v,
             adj1_v, c2_v, s2_v,
             buf0, buf1, stage, sem0, sem1, sem2):
    w = lax.axis_index("s") * NC + lax.axis_index("c")
    iota = lax.iota(jnp.int32, 16)


    pltpu.sync_copy(nodes_hbm.at[pl.ds(w * NB_W, NB_W)], nodes_v)

    def nsplit_body(i, carry):
        nv = nodes_v[pl.ds(i * 16, 16)]
        nd4_v[pl.ds(i * 16, 16)] = nv // 4
        nm4_v[pl.ds(i * 16, 16)] = (nv % 4) * 32
        return carry

    lax.fori_loop(0, NB_W // 16, nsplit_body, 0)
    pltpu.async_copy(adj_hbm.at[nd4_v], adj0_v, sem2).wait()
    pltpu.sync_copy(c1_hbm.at[pl.ds(w * S1_W, S1_W)], c1_v)

    def s1_body(i, carry):
        e = i * 16 + iota
        r = e // S1_PER
        cv = c1_v[pl.ds(i * 16, 16)] + plsc.load_gather(nm4_v, [r])
        s1_v[pl.ds(i * 16, 16)] = plsc.load_gather(adj0_v, [r, cv])
        return carry

    lax.fori_loop(0, S1_W // 16, s1_body, 0)

    def s1split_body(i, carry):
        sv = s1_v[pl.ds(i * 16, 16)]
        s1d4_v[pl.ds(i * 16, 16)] = sv // 4
        s1m4_v[pl.ds(i * 16, 16)] = (sv % 4) * 32
        return carry

    lax.fori_loop(0, S1_W // 16, s1split_body, 0)

    pltpu.async_copy(feat_hbm.at[nodes_v], buf0.at[pl.ds(0, NB_W)], sem0)
    pltpu.async_copy(feat_hbm.at[s1_v.at[pl.ds(0, H1_CHUNK)]],
                     buf1.at[pl.ds(0, H1_CHUNK)], sem1)

    def s2_quarter(q, carry):
        pltpu.async_copy(adj_hbm.at[s1d4_v.at[pl.ds(q * Q_PAR, Q_PAR)]],
                         adj1_v, sem2).wait()
        pltpu.sync_copy(c2_hbm.at[pl.ds(w * S2_W + q * Q_S2, Q_S2)], c2_v)

        def s2_body(i, inner):
            e = q * Q_S2 + i * 16 + iota
            r = e // S2_PER
            cv = c2_v[pl.ds(i * 16, 16)] + plsc.load_gather(s1m4_v, [r])
            s2_v[pl.ds(q * Q_S2 + i * 16, 16)] = plsc.load_gather(
                adj1_v, [r - q * Q_PAR, cv])
            return inner

        lax.fori_loop(0, Q_S2 // 16, s2_body, 0)
        return carry

    lax.fori_loop(0, 4, s2_quarter, 0)

    def fire(g, buf, sem):
        off = g * G_ROWS
        pltpu.async_copy(feat_hbm.at[s2_v.at[pl.ds(off, 128)]],
                         buf.at[pl.ds(0, 128)], sem)
        pltpu.async_copy(feat_hbm.at[s2_v.at[pl.ds(off + 128, G_ROWS - 128)]],
                         buf.at[pl.ds(128, G_ROWS - 128)], sem)

    def drain(buf, sem):
        pltpu.make_async_copy(feat_hbm.at[pl.ds(0, G_ROWS)], buf, sem).wait()

    bufs = (buf0, buf1)
    sems = (sem0, sem1)

    def h1_fire(k):
        b = (k + 1) % 2
        pltpu.async_copy(feat_hbm.at[s1_v.at[pl.ds(k * H1_CHUNK, H1_CHUNK)]],
                         bufs[b].at[pl.ds(0, H1_CHUNK)], sems[b])

    def h1_drain_out(k):
        b = (k + 1) % 2
        pltpu.make_async_copy(feat_hbm.at[pl.ds(0, H1_CHUNK)],
                              bufs[b].at[pl.ds(0, H1_CHUNK)],
                              sems[b]).wait()
        pltpu.sync_copy(bufs[b].at[pl.ds(0, H1_CHUNK)],
                        h1_hbm.at[pl.ds(w * S1_W + k * H1_CHUNK, H1_CHUNK)])

    pltpu.make_async_copy(feat_hbm.at[pl.ds(0, NB_W)],
                          buf0.at[pl.ds(0, NB_W)], sem0).wait()
    pltpu.sync_copy(buf0.at[pl.ds(0, NB_W)], h0_hbm.at[pl.ds(w * NB_W, NB_W)])
    h1_fire(1)
    h1_drain_out(0)
    h1_fire(2)
    h1_drain_out(1)
    h1_fire(3)
    h1_drain_out(2)
    fire(0, buf1, sem1)
    h1_drain_out(3)
    fire(1, buf0, sem0)

    def process(g, buf):
        def par_body(p, carry):
            base = p * S2_PER

            def row_body(r, acc):
                row = base + r
                return tuple(acc[c] + buf[row, pl.ds(c * 16, 16)]
                             for c in range(16))

            acc0 = tuple(jnp.zeros((16,), jnp.float32) for _ in range(16))
            acc = lax.fori_loop(0, S2_PER, row_body, acc0)
            for c in range(16):
                stage[p, pl.ds(c * 16, 16)] = acc[c] * (1.0 / S2_PER)
            return carry

        lax.fori_loop(0, GP, par_body, 0)
        pltpu.sync_copy(stage, m2_hbm.at[pl.ds(w * S1_W + g * GP, GP)])

    def m2_body(t, carry):
        g0 = 2 * t
        drain(buf1, sem1)
        process(g0, buf1)

        @pl.when(t < NG // 2 - 1)
        def _():
            fire(g0 + 2, buf1, sem1)

        drain(buf0, sem0)
        process(g0 + 1, buf0)

        @pl.when(t < NG // 2 - 1)
        def _():
            fire(g0 + 3, buf0, sem0)

        return carry

    lax.fori_loop(0, NG // 2, m2_body, 0)


_sc_gather = functools.partial(
    pl.kernel,
    out_type=(
        jax.ShapeDtypeStruct((B, D), jnp.float32),
        jax.ShapeDtypeStruct((S1, D), jnp.float32),
        jax.ShapeDtypeStruct((S1, D), jnp.float32),
    ),
    mesh=_sc_mesh,
    compiler_params=pltpu.CompilerParams(needs_layout_passes=False, use_tc_tiling_on_sc=True),
    scratch_types=[
        pltpu.VMEM((NB_W,), jnp.int32),
        pltpu.VMEM((NB_W,), jnp.int32),
        pltpu.VMEM((NB_W,), jnp.int32),
        pltpu.VMEM((NB_W, 128), jnp.int32),
        pltpu.VMEM((S1_W,), jnp.int32),
        pltpu.VMEM((S1_W,), jnp.int32),
        pltpu.VMEM((S1_W,), jnp.int32),
        pltpu.VMEM((S1_W,), jnp.int32),
        pltpu.VMEM((Q_PAR, 128), jnp.int32),
        pltpu.VMEM((Q_S2,), jnp.int32),
        pltpu.VMEM((S2_W,), jnp.int32),
        pltpu.VMEM((G_ROWS, D), jnp.float32),
        pltpu.VMEM((G_ROWS, D), jnp.float32),
        pltpu.VMEM((GP, D), jnp.float32),
        pltpu.SemaphoreType.DMA,
        pltpu.SemaphoreType.DMA,
        pltpu.SemaphoreType.DMA,
    ],
)(_sc_body)


T1_ROWS = 1280
T1_OUT = T1_ROWS // S1_PER
T1_STEPS = S1 // T1_ROWS


def _tc_body(h1_ref, m2_ref, h0_ref, w1_ref, w2_ref, fcw_ref, fcb_ref,
             out_ref, a1m_s, h1m_s):
    i = pl.program_id(0)
    h1b = h1_ref[...]
    w1 = w1_ref[...]
    x = (jnp.dot(h1b, w1[:D], preferred_element_type=jnp.float32)
         + jnp.dot(m2_ref[...], w1[D:], preferred_element_type=jnp.float32))
    x = jnp.maximum(x, 0.0)
    nrm = jnp.sqrt(jnp.sum(x * x, axis=1, keepdims=True))
    a1 = x / (nrm + 1e-8)
    a1m_s[pl.ds(i * T1_OUT, T1_OUT), :] = jnp.mean(
        a1.reshape(T1_OUT, S1_PER, D), axis=1)
    h1m_s[pl.ds(i * T1_OUT, T1_OUT), :] = jnp.mean(
        h1b.reshape(T1_OUT, S1_PER, D), axis=1)

    @pl.when(i == T1_STEPS - 1)
    def _():
        x0 = (jnp.dot(h0_ref[...], w1[:D], preferred_element_type=jnp.float32)
              + jnp.dot(h1m_s[...], w1[D:], preferred_element_type=jnp.float32))
        x0 = jnp.maximum(x0, 0.0)
        a0 = x0 / (jnp.sqrt(jnp.sum(x0 * x0, axis=1, keepdims=True)) + 1e-8)
        w2 = w2_ref[...]
        y = (jnp.dot(a0, w2[:D], preferred_element_type=jnp.float32)
             + jnp.dot(a1m_s[...], w2[D:], preferred_element_type=jnp.float32))
        y = jnp.maximum(y, 0.0)
        b0 = y / (jnp.sqrt(jnp.sum(y * y, axis=1, keepdims=True)) + 1e-8)
        out_ref[...] = (jnp.dot(b0, fcw_ref[...],
                                preferred_element_type=jnp.float32)
                        + fcb_ref[...])


def _tc(h1, m2, h0, W1, W2, fc_w, fc_b):
    return pl.pallas_call(
        _tc_body,
        grid=(T1_STEPS,),
        in_specs=[
            pl.BlockSpec((T1_ROWS, D), lambda i: (i, 0)),
            pl.BlockSpec((T1_ROWS, D), lambda i: (i, 0)),
            pl.BlockSpec((B, D), lambda i: (0, 0)),
            pl.BlockSpec((2 * D, D), lambda i: (0, 0)),
            pl.BlockSpec((2 * D, D), lambda i: (0, 0)),
            pl.BlockSpec((D, N_CLS), lambda i: (0, 0)),
            pl.BlockSpec((1, N_CLS), lambda i: (0, 0)),
        ],
        out_specs=pl.BlockSpec((B, N_CLS), lambda i: (0, 0)),
        out_shape=jax.ShapeDtypeStruct((B, N_CLS), jnp.float32),
        scratch_shapes=[
            pltpu.VMEM((B, D), jnp.float32),
            pltpu.VMEM((B, D), jnp.float32),
        ],
    )(h1, m2, h0, W1, W2, fc_w, fc_b.reshape(1, -1))


def kernel(nodes, features, adj, W1, W2, fc_w, fc_b):
    nodes = nodes.astype(jnp.int32)
    max_deg = adj.shape[1]
    adj = adj.astype(jnp.int32).reshape(-1, 128)
    key = jax.random.key(42)
    key, sub = jax.random.split(key)
    cols1 = jax.random.randint(sub, (B, S1_PER), 0, max_deg)
    key, sub = jax.random.split(key)
    cols2 = jax.random.randint(sub, (S1, S2_PER), 0, max_deg)
    c1 = cols1.reshape(-1).astype(jnp.int32)
    c2 = cols2.reshape(-1).astype(jnp.int32)

    h0, h1, m2 = _sc_gather(nodes, adj, features, c1, c2)
    return _tc(h1, m2, h0, W1, W2, fc_w, fc_b)
